# baseline (speedup 1.0000x reference)
import functools

import jax
import jax.numpy as jnp
from jax import lax
from jax.experimental import pallas as pl
from jax.experimental.pallas import tpu as pltpu
from jax.experimental.pallas import tpu_sc as plsc

B, T = 4096, 200
D = 64
B_TOTAL = B * T
CHUNK = 128
NBUF = 4

_cache = {}


def _build():
    if "k" in _cache:
        return _cache["k"]
    info = plsc.get_sparse_core_info()
    NC, NS = info.num_cores, info.num_subcores
    NW = NC * NS
    n_chunks = B_TOTAL // CHUNK
    chunks_per_w = n_chunks // NW
    n_groups = chunks_per_w // NBUF
    mesh = plsc.VectorSubcoreMesh(core_axis_name="c", subcore_axis_name="s")

    @functools.partial(
        pl.kernel,
        mesh=mesh,
        compiler_params=pltpu.CompilerParams(use_tc_tiling_on_sc=False),
        out_type=jax.ShapeDtypeStruct((B_TOTAL, D), jnp.float32),
        scratch_types=[
            pltpu.VMEM((chunks_per_w, CHUNK), jnp.int32),
            pltpu.VMEM((NBUF, CHUNK, D), jnp.float32),
            pltpu.SemaphoreType.DMA,
        ],
    )
    def emb(idx_hbm, table_hbm, out_hbm, idx_v, rows_v, gsem):
        wid = lax.axis_index("s") * NC + lax.axis_index("c")
        cbase = wid * chunks_per_w
        pltpu.sync_copy(idx_hbm.at[pl.ds(cbase, chunks_per_w)], idx_v)

        def group(gi, carry):
            g0 = gi * NBUF
            copies = []
            for b in range(NBUF):
                copies.append(
                    pltpu.async_copy(
                        table_hbm.at[idx_v.at[g0 + b]], rows_v.at[b], gsem
                    )
                )
            for b in range(NBUF):
                copies[b].wait()
            for b in range(NBUF):
                row0 = (cbase + g0 + b) * CHUNK
                pltpu.sync_copy(rows_v.at[b], out_hbm.at[pl.ds(row0, CHUNK)])
            return carry

        lax.fori_loop(0, n_groups, group, 0)

    _cache["k"] = emb
    return emb


def kernel(idx_texts, table):
    idx_flat = idx_texts.reshape(B_TOTAL // CHUNK, CHUNK).astype(jnp.int32)
    out = _build()(idx_flat, table)
    return out.reshape(B, T, D)


# speedup vs baseline: 1.0308x; 1.0308x over previous
import functools

import jax
import jax.numpy as jnp
from jax import lax
from jax.experimental import pallas as pl
from jax.experimental.pallas import tpu as pltpu
from jax.experimental.pallas import tpu_sc as plsc

B, T = 4096, 200
D = 64
B_TOTAL = B * T
CHUNK = 128
NBUF = 4

_cache = {}


def _build():
    if "k" in _cache:
        return _cache["k"]
    info = plsc.get_sparse_core_info()
    NC, NS = info.num_cores, info.num_subcores
    NW = NC * NS
    n_chunks = B_TOTAL // CHUNK
    chunks_per_w = n_chunks // NW
    n_groups = chunks_per_w // NBUF
    assert n_groups % 2 == 0 and n_groups >= 4
    grows = NBUF * CHUNK
    mesh = plsc.VectorSubcoreMesh(core_axis_name="c", subcore_axis_name="s")

    @functools.partial(
        pl.kernel,
        mesh=mesh,
        compiler_params=pltpu.CompilerParams(use_tc_tiling_on_sc=False),
        out_type=jax.ShapeDtypeStruct((B_TOTAL, D), jnp.float32),
        scratch_types=[
            pltpu.VMEM((chunks_per_w, CHUNK), jnp.int32),
            pltpu.VMEM((2, grows, D), jnp.float32),
            pltpu.SemaphoreType.DMA,
            pltpu.SemaphoreType.DMA,
            pltpu.SemaphoreType.DMA,
            pltpu.SemaphoreType.DMA,
        ],
    )
    def emb(idx_hbm, table_hbm, out_hbm, idx_v, rows_v, g0sem, g1sem, s0sem, s1sem):
        wid = lax.axis_index("s") * NC + lax.axis_index("c")
        cbase = wid * chunks_per_w
        gsems = [g0sem, g1sem]
        ssems = [s0sem, s1sem]
        pltpu.sync_copy(idx_hbm.at[pl.ds(cbase, chunks_per_w)], idx_v)

        def start_gathers(gi, s):
            for b in range(NBUF):
                pltpu.async_copy(
                    table_hbm.at[idx_v.at[gi * NBUF + b]],
                    rows_v.at[s].at[pl.ds(b * CHUNK, CHUNK)],
                    gsems[s],
                )

        def wait_gathers(s):
            for b in range(NBUF):
                pltpu.make_async_copy(
                    table_hbm.at[idx_v.at[b]],
                    rows_v.at[s].at[pl.ds(b * CHUNK, CHUNK)],
                    gsems[s],
                ).wait()

        def start_store(gi, s):
            pltpu.async_copy(
                rows_v.at[s],
                out_hbm.at[pl.ds((cbase + gi * NBUF) * CHUNK, grows)],
                ssems[s],
            )

        def wait_store(s):
            pltpu.make_async_copy(
                rows_v.at[s],
                out_hbm.at[pl.ds(cbase * CHUNK, grows)],
                ssems[s],
            ).wait()

        start_gathers(0, 0)
        start_gathers(1, 1)
        wait_gathers(0)
        start_store(0, 0)

        def pair(p, carry):
            go = 2 * p + 1
            wait_store(0)
            start_gathers(go + 1, 0)
            wait_gathers(1)
            start_store(go, 1)
            ge = go + 1
            wait_store(1)
            start_gathers(ge + 1, 1)
            wait_gathers(0)
            start_store(ge, 0)
            return carry

        lax.fori_loop(0, (n_groups - 2) // 2, pair, 0)

        wait_gathers(1)
        start_store(n_groups - 1, 1)
        wait_store(0)
        wait_store(1)

    _cache["k"] = emb
    return emb


def kernel(idx_texts, table):
    idx_flat = idx_texts.reshape(B_TOTAL // CHUNK, CHUNK).astype(jnp.int32)
    out = _build()(idx_flat, table)
    return out.reshape(B, T, D)
